# baseline (speedup 1.0000x reference)
import jax
import jax.numpy as jnp
from jax import lax
from jax.experimental import pallas as pl
from jax.experimental.pallas import tpu as pltpu
from jax.experimental.pallas import tpu_sc as plsc

D = 128
DH = D // 2
NC, NS = 2, 16
NPAD = 10240
ZSTR = NPAD // NS
CH = 128
NB = 4

_mesh = plsc.VectorSubcoreMesh(
    core_axis_name="c", subcore_axis_name="s", num_cores=NC, num_subcores=NS
)
_params = pltpu.CompilerParams(
    needs_layout_passes=False, use_tc_tiling_on_sc=False
)


def _worker():
    return lax.axis_index("c"), lax.axis_index("s")


def _fill_1d(ref, n, val):
    v16 = jnp.full((16,), val, jnp.float32)

    @pl.loop(0, n // 16)
    def _(i):
        ref[pl.ds(i * 16, 16)] = v16


def _make_scatter_kernel(nchunk, gcol, with_counts):
    ngroups = nchunk // NB
    base_g = ngroups // NS
    rem_g = ngroups - base_g * NS
    outs = [
        jax.ShapeDtypeStruct((NPAD, DH), jnp.float32),
        jax.ShapeDtypeStruct((NPAD, DH), jnp.float32),
    ]
    if with_counts:
        outs += [
            jax.ShapeDtypeStruct((NPAD,), jnp.float32),
        ]
    scratch = (
        [pltpu.VMEM_SHARED((NPAD, DH), jnp.float32)]
        + [pltpu.VMEM_SHARED((NPAD, DH), jnp.float32)]
        + [pltpu.VMEM((2, NB, CH), jnp.int32)]
        + [pltpu.VMEM((CH, DH), jnp.float32) for _ in range(NB)]
        + [pltpu.SemaphoreType.DMA]
        + [pltpu.SemaphoreType.DMA for _ in range(NB)]
        + [pltpu.SemaphoreType.DMA for _ in range(NB)]
    )
    if with_counts:
        scratch += [
            pltpu.VMEM_SHARED((NPAD,), jnp.float32),
            pltpu.VMEM_SHARED((NPAD,), jnp.float32),
            pltpu.VMEM((CH,), jnp.float32),
            pltpu.VMEM((ZSTR,), jnp.float32),
            pltpu.SemaphoreType.DMA,
            pltpu.SemaphoreType.DMA,
        ]

    def body(src0_hbm, src1_hbm, eidx_hbm, *rest):
        if with_counts:
            (half0, half1, dinv_out, acc, xsp, pix, *rest2) = rest
            rows = rest2[:NB]
            isem = rest2[NB]
            gsems = rest2[NB + 1:2 * NB + 1]
            ssems = rest2[2 * NB + 1:3 * NB + 1]
            cnt_d, cnt_b, ones, zcnt, dsem, bsem = rest2[3 * NB + 1:]
        else:
            (half0, half1, acc, xsp, pix, *rest2) = rest
            rows = rest2[:NB]
            isem = rest2[NB]
            gsems = rest2[NB + 1:2 * NB + 1]
            ssems = rest2[2 * NB + 1:3 * NB + 1]
        c, s = _worker()
        z16 = jnp.zeros((16,), jnp.float32)

        @pl.loop(0, CH * (DH // 16))
        def _(t):
            rows[0][t // (DH // 16), pl.ds((t % (DH // 16)) * 16, 16)] = z16

        @pl.loop(0, ZSTR // CH)
        def _(q):
            pltpu.sync_copy(rows[0], acc.at[pl.ds(s * ZSTR + q * CH, CH)])

        if with_counts:
            _fill_1d(ones, CH, 1.0)
            _fill_1d(zcnt, ZSTR, 0.0)
            pltpu.sync_copy(zcnt, cnt_d.at[pl.ds(s * ZSTR, ZSTR)])
            pltpu.sync_copy(zcnt, cnt_b.at[pl.ds(s * ZSTR, ZSTR)])

        stl = pl.ds(s * ZSTR, ZSTR)

        @pl.when(c == 0)
        def _():
            pltpu.sync_copy(src0_hbm.at[stl], xsp.at[stl])

        @pl.when(c == 1)
        def _():
            pltpu.sync_copy(src1_hbm.at[stl], xsp.at[stl])

        plsc.subcore_barrier()

        def do_group(cb, nb):
            d0 = pltpu.async_copy(eidx_hbm.at[0, pl.ds(cb, nb)],
                                  pix.at[0], isem)
            d1 = pltpu.async_copy(eidx_hbm.at[1, pl.ds(cb, nb)],
                                  pix.at[1], isem)
            d0.wait()
            d1.wait()
            dgs = []
            for b in range(nb):
                if b % 2 == 0:
                    dgs.append(
                        pltpu.async_copy(xsp.at[pix.at[gcol, b]], rows[b],
                                         gsems[b])
                    )
                else:
                    @pl.when(c == 0)
                    def _(b=b):
                        pltpu.async_copy(src0_hbm.at[pix.at[gcol, b]],
                                         rows[b], gsems[b])

                    @pl.when(c == 1)
                    def _(b=b):
                        pltpu.async_copy(src1_hbm.at[pix.at[gcol, b]],
                                         rows[b], gsems[b])
                    dgs.append(
                        pltpu.make_async_copy(src0_hbm.at[pix.at[gcol, b]],
                                              rows[b], gsems[b])
                    )
            dcs = []
            if with_counts:
                for b in range(nb):
                    dcs.append(
                        pltpu.async_copy(ones, cnt_d.at[pix.at[0, b]],
                                         dsem, add=True)
                    )
                    dcs.append(
                        pltpu.async_copy(ones, cnt_b.at[pix.at[1, b]],
                                         bsem, add=True)
                    )
            dss = []
            for b in range(nb):
                dgs[b].wait()
                dss.append(
                    pltpu.async_copy(
                        rows[b], acc.at[pix.at[1 - gcol, b]], ssems[b],
                        add=True,
                    )
                )
            for d in dss:
                d.wait()
            for d in dcs:
                d.wait()

        @pl.loop(0, base_g)
        def _(gq):
            do_group(NB * (gq * NS + s), NB)

        if rem_g:
            @pl.when(s < rem_g)
            def _():
                do_group(NB * (base_g * NS + s), NB)

        plsc.subcore_barrier()

        sl = pl.ds(s * ZSTR, ZSTR)
        if with_counts:

            def _recip_inplace():
                @pl.loop(0, ZSTR // 16)
                def _(i):
                    slv = pl.ds(i * 16, 16)
                    v = zcnt[slv]
                    zcnt[slv] = jnp.where(v > 0.0, 1.0 / v, 0.0)

            @pl.when(c == 0)
            def _():
                pltpu.sync_copy(cnt_d.at[sl], zcnt)
                _recip_inplace()
                pltpu.sync_copy(zcnt, dinv_out.at[sl])

            pltpu.sync_copy(cnt_b.at[sl], zcnt)
            _recip_inplace()
            zi = jnp.zeros((16,), jnp.int32)

            @pl.loop(0, ZSTR // CH)
            def _(q):
                rsl = pl.ds(s * ZSTR + q * CH, CH)
                pltpu.sync_copy(acc.at[rsl], rows[0])

                @pl.loop(0, CH)
                def _(i):
                    bs = plsc.load_gather(zcnt, [zi + (q * CH + i)])
                    for k in range(DH // 16):
                        ksl = pl.ds(k * 16, 16)
                        rows[0][i, ksl] = rows[0][i, ksl] * bs

                @pl.when(c == 0)
                def _():
                    pltpu.sync_copy(rows[0], half0.at[rsl])

                @pl.when(c == 1)
                def _():
                    pltpu.sync_copy(rows[0], half1.at[rsl])
        else:

            @pl.when(c == 0)
            def _():
                pltpu.sync_copy(acc.at[sl], half0.at[sl])

            @pl.when(c == 1)
            def _():
                pltpu.sync_copy(acc.at[sl], half1.at[sl])

    return pl.kernel(
        body,
        out_type=tuple(outs) if with_counts else tuple(outs),
        mesh=_mesh,
        scratch_types=scratch,
        compiler_params=_params,
    )


def _mm_body(x_ref, w0_ref, w1_ref, o0_ref, o1_ref):
    xp = x_ref[...]
    o0_ref[...] = jnp.dot(xp, w0_ref[...], preferred_element_type=jnp.float32)
    o1_ref[...] = jnp.dot(xp, w1_ref[...], preferred_element_type=jnp.float32)


def _out_body(q0_ref, q1_ref, dinv_ref, b_ref, o_ref):
    r = jnp.concatenate([q0_ref[...], q1_ref[...]], axis=1)
    r = r * dinv_ref[...] + b_ref[...]
    m = jnp.max(r, axis=1, keepdims=True)
    e = jnp.exp(r - m)
    o_ref[...] = e / jnp.sum(e, axis=1, keepdims=True)


def kernel(X, edge_index, W, bias):
    n, d_in = X.shape
    d_out = W.shape[1]
    e = edge_index.shape[1]
    nchunk = e // CH

    Xp2 = jnp.concatenate(
        [X, jnp.zeros((NPAD - n, d_in), jnp.float32)], axis=0
    ).reshape(NPAD // 2, 2 * d_in)
    z = jnp.zeros((d_in, DH), jnp.float32)
    wp0 = jnp.concatenate(
        [jnp.concatenate([W[:, :DH], z], axis=1),
         jnp.concatenate([z, W[:, :DH]], axis=1)], axis=0)
    wp1 = jnp.concatenate(
        [jnp.concatenate([W[:, DH:], z], axis=1),
         jnp.concatenate([z, W[:, DH:]], axis=1)], axis=0)

    rb = 1024
    x0p, x1p = pl.pallas_call(
        _mm_body,
        grid=(NPAD // rb,),
        in_specs=[
            pl.BlockSpec((rb // 2, 2 * d_in), lambda i: (i, 0)),
            pl.BlockSpec((2 * d_in, d_out), lambda i: (0, 0)),
            pl.BlockSpec((2 * d_in, d_out), lambda i: (0, 0)),
        ],
        out_specs=[
            pl.BlockSpec((rb // 2, D), lambda i: (i, 0)),
            pl.BlockSpec((rb // 2, D), lambda i: (i, 0)),
        ],
        out_shape=[
            jax.ShapeDtypeStruct((NPAD // 2, D), jnp.float32),
            jax.ShapeDtypeStruct((NPAD // 2, D), jnp.float32),
        ],
    )(Xp2, wp0, wp1)
    x0 = x0p.reshape(NPAD, DH)
    x1 = x1p.reshape(NPAD, DH)

    ei3 = edge_index.reshape(2, nchunk, CH)

    ef0, ef1, dinv = _make_scatter_kernel(nchunk, 0, True)(x0, x1, ei3)

    oh0, oh1 = _make_scatter_kernel(nchunk, 1, False)(ef0, ef1, ei3)

    ob = 1000
    dinv_col = dinv.reshape(NPAD, 1)
    bias2 = bias.reshape(1, d_out)
    out = pl.pallas_call(
        _out_body,
        grid=(n // ob,),
        in_specs=[
            pl.BlockSpec((ob, DH), lambda i: (i, 0)),
            pl.BlockSpec((ob, DH), lambda i: (i, 0)),
            pl.BlockSpec((ob, 1), lambda i: (i, 0)),
            pl.BlockSpec((1, d_out), lambda i: (0, 0)),
        ],
        out_specs=pl.BlockSpec((ob, d_out), lambda i: (i, 0)),
        out_shape=jax.ShapeDtypeStruct((n, d_out), jnp.float32),
    )(oh0, oh1, dinv_col, bias2)
    return out


# speedup vs baseline: 1.0128x; 1.0128x over previous
import jax
import jax.numpy as jnp
from jax import lax
from jax.experimental import pallas as pl
from jax.experimental.pallas import tpu as pltpu
from jax.experimental.pallas import tpu_sc as plsc

D = 128
DH = D // 2
NC, NS = 2, 16
NPAD = 10240
ZSTR = NPAD // NS
CH = 128
NB = 4

_mesh = plsc.VectorSubcoreMesh(
    core_axis_name="c", subcore_axis_name="s", num_cores=NC, num_subcores=NS
)
_params = pltpu.CompilerParams(
    needs_layout_passes=False, use_tc_tiling_on_sc=False
)


def _worker():
    return lax.axis_index("c"), lax.axis_index("s")


def _fill_1d(ref, n, val):
    v16 = jnp.full((16,), val, jnp.float32)

    @pl.loop(0, n // 16)
    def _(i):
        ref[pl.ds(i * 16, 16)] = v16


def _make_scatter_kernel(nchunk, gcol, with_counts):
    ngroups = nchunk // NB
    base_g = ngroups // NS
    rem_c = nchunk - base_g * NS * NB
    outs = [
        jax.ShapeDtypeStruct((NPAD, DH), jnp.float32),
        jax.ShapeDtypeStruct((NPAD, DH), jnp.float32),
    ]
    if with_counts:
        outs += [
            jax.ShapeDtypeStruct((NPAD,), jnp.float32),
        ]
    scratch = (
        [pltpu.VMEM_SHARED((NPAD, DH), jnp.float32)]
        + [pltpu.VMEM_SHARED((NPAD, DH), jnp.float32)]
        + [pltpu.VMEM((2, NB, CH), jnp.int32)]
        + [pltpu.VMEM((CH, DH), jnp.float32) for _ in range(NB)]
        + [pltpu.SemaphoreType.DMA]
        + [pltpu.SemaphoreType.DMA for _ in range(NB)]
        + [pltpu.SemaphoreType.DMA for _ in range(NB)]
    )
    if with_counts:
        scratch += [
            pltpu.VMEM_SHARED((NPAD,), jnp.float32),
            pltpu.VMEM_SHARED((NPAD,), jnp.float32),
            pltpu.VMEM((CH,), jnp.float32),
            pltpu.VMEM((ZSTR,), jnp.float32),
            pltpu.SemaphoreType.DMA,
            pltpu.SemaphoreType.DMA,
        ]

    def body(src0_hbm, src1_hbm, eidx_hbm, *rest):
        if with_counts:
            (half0, half1, dinv_out, acc, xsp, pix, *rest2) = rest
            rows = rest2[:NB]
            isem = rest2[NB]
            gsems = rest2[NB + 1:2 * NB + 1]
            ssems = rest2[2 * NB + 1:3 * NB + 1]
            cnt_d, cnt_b, ones, zcnt, dsem, bsem = rest2[3 * NB + 1:]
        else:
            (half0, half1, acc, xsp, pix, *rest2) = rest
            rows = rest2[:NB]
            isem = rest2[NB]
            gsems = rest2[NB + 1:2 * NB + 1]
            ssems = rest2[2 * NB + 1:3 * NB + 1]
        c, s = _worker()
        z16 = jnp.zeros((16,), jnp.float32)

        @pl.loop(0, CH * (DH // 16))
        def _(t):
            rows[0][t // (DH // 16), pl.ds((t % (DH // 16)) * 16, 16)] = z16

        @pl.loop(0, ZSTR // CH)
        def _(q):
            pltpu.sync_copy(rows[0], acc.at[pl.ds(s * ZSTR + q * CH, CH)])

        if with_counts:
            _fill_1d(ones, CH, 1.0)
            _fill_1d(zcnt, ZSTR, 0.0)
            pltpu.sync_copy(zcnt, cnt_d.at[pl.ds(s * ZSTR, ZSTR)])
            pltpu.sync_copy(zcnt, cnt_b.at[pl.ds(s * ZSTR, ZSTR)])

        stl = pl.ds(s * ZSTR, ZSTR)

        @pl.when(c == 0)
        def _():
            pltpu.sync_copy(src0_hbm.at[stl], xsp.at[stl])

        @pl.when(c == 1)
        def _():
            pltpu.sync_copy(src1_hbm.at[stl], xsp.at[stl])

        plsc.subcore_barrier()

        def do_group(cb, nb):
            d0 = pltpu.async_copy(eidx_hbm.at[0, pl.ds(cb, nb)],
                                  pix.at[0, pl.ds(0, nb)], isem)
            d1 = pltpu.async_copy(eidx_hbm.at[1, pl.ds(cb, nb)],
                                  pix.at[1, pl.ds(0, nb)], isem)
            d0.wait()
            d1.wait()
            dgs = []
            for b in range(nb):
                if b % 2 == 0:
                    dgs.append(
                        pltpu.async_copy(xsp.at[pix.at[gcol, b]], rows[b],
                                         gsems[b])
                    )
                else:
                    @pl.when(c == 0)
                    def _(b=b):
                        pltpu.async_copy(src0_hbm.at[pix.at[gcol, b]],
                                         rows[b], gsems[b])

                    @pl.when(c == 1)
                    def _(b=b):
                        pltpu.async_copy(src1_hbm.at[pix.at[gcol, b]],
                                         rows[b], gsems[b])
                    dgs.append(
                        pltpu.make_async_copy(src0_hbm.at[pix.at[gcol, b]],
                                              rows[b], gsems[b])
                    )
            dcs = []
            if with_counts:
                for b in range(nb):
                    dcs.append(
                        pltpu.async_copy(ones, cnt_d.at[pix.at[0, b]],
                                         dsem, add=True)
                    )
                    dcs.append(
                        pltpu.async_copy(ones, cnt_b.at[pix.at[1, b]],
                                         bsem, add=True)
                    )
            dss = []
            for b in range(nb):
                dgs[b].wait()
                dss.append(
                    pltpu.async_copy(
                        rows[b], acc.at[pix.at[1 - gcol, b]], ssems[b],
                        add=True,
                    )
                )
            for d in dss:
                d.wait()
            for d in dcs:
                d.wait()

        @pl.loop(0, base_g)
        def _(gq):
            do_group(NB * (gq * NS + s), NB)

        if rem_c:
            @pl.when(s < rem_c)
            def _():
                do_group(NB * base_g * NS + s, 1)

        plsc.subcore_barrier()

        sl = pl.ds(s * ZSTR, ZSTR)
        if with_counts:

            def _recip_inplace():
                @pl.loop(0, ZSTR // 16)
                def _(i):
                    slv = pl.ds(i * 16, 16)
                    v = zcnt[slv]
                    zcnt[slv] = jnp.where(v > 0.0, 1.0 / v, 0.0)

            @pl.when(c == 0)
            def _():
                pltpu.sync_copy(cnt_d.at[sl], zcnt)
                _recip_inplace()
                pltpu.sync_copy(zcnt, dinv_out.at[sl])

            pltpu.sync_copy(cnt_b.at[sl], zcnt)
            _recip_inplace()
            zi = jnp.zeros((16,), jnp.int32)

            @pl.loop(0, ZSTR // CH)
            def _(q):
                rsl = pl.ds(s * ZSTR + q * CH, CH)
                pltpu.sync_copy(acc.at[rsl], rows[0])

                @pl.loop(0, CH)
                def _(i):
                    bs = plsc.load_gather(zcnt, [zi + (q * CH + i)])
                    for k in range(DH // 16):
                        ksl = pl.ds(k * 16, 16)
                        rows[0][i, ksl] = rows[0][i, ksl] * bs

                @pl.when(c == 0)
                def _():
                    pltpu.sync_copy(rows[0], half0.at[rsl])

                @pl.when(c == 1)
                def _():
                    pltpu.sync_copy(rows[0], half1.at[rsl])
        else:

            @pl.when(c == 0)
            def _():
                pltpu.sync_copy(acc.at[sl], half0.at[sl])

            @pl.when(c == 1)
            def _():
                pltpu.sync_copy(acc.at[sl], half1.at[sl])

    return pl.kernel(
        body,
        out_type=tuple(outs) if with_counts else tuple(outs),
        mesh=_mesh,
        scratch_types=scratch,
        compiler_params=_params,
    )


def _mm_body(x_ref, w0_ref, w1_ref, o0_ref, o1_ref):
    xp = x_ref[...]
    o0_ref[...] = jnp.dot(xp, w0_ref[...], preferred_element_type=jnp.float32)
    o1_ref[...] = jnp.dot(xp, w1_ref[...], preferred_element_type=jnp.float32)


def _out_body(q0_ref, q1_ref, dinv_ref, b_ref, o_ref):
    r = jnp.concatenate([q0_ref[...], q1_ref[...]], axis=1)
    r = r * dinv_ref[...] + b_ref[...]
    m = jnp.max(r, axis=1, keepdims=True)
    e = jnp.exp(r - m)
    o_ref[...] = e / jnp.sum(e, axis=1, keepdims=True)


def kernel(X, edge_index, W, bias):
    n, d_in = X.shape
    d_out = W.shape[1]
    e = edge_index.shape[1]
    nchunk = e // CH

    Xp2 = jnp.concatenate(
        [X, jnp.zeros((NPAD - n, d_in), jnp.float32)], axis=0
    ).reshape(NPAD // 2, 2 * d_in)
    z = jnp.zeros((d_in, DH), jnp.float32)
    wp0 = jnp.concatenate(
        [jnp.concatenate([W[:, :DH], z], axis=1),
         jnp.concatenate([z, W[:, :DH]], axis=1)], axis=0)
    wp1 = jnp.concatenate(
        [jnp.concatenate([W[:, DH:], z], axis=1),
         jnp.concatenate([z, W[:, DH:]], axis=1)], axis=0)

    rb = 1024
    x0p, x1p = pl.pallas_call(
        _mm_body,
        grid=(NPAD // rb,),
        in_specs=[
            pl.BlockSpec((rb // 2, 2 * d_in), lambda i: (i, 0)),
            pl.BlockSpec((2 * d_in, d_out), lambda i: (0, 0)),
            pl.BlockSpec((2 * d_in, d_out), lambda i: (0, 0)),
        ],
        out_specs=[
            pl.BlockSpec((rb // 2, D), lambda i: (i, 0)),
            pl.BlockSpec((rb // 2, D), lambda i: (i, 0)),
        ],
        out_shape=[
            jax.ShapeDtypeStruct((NPAD // 2, D), jnp.float32),
            jax.ShapeDtypeStruct((NPAD // 2, D), jnp.float32),
        ],
    )(Xp2, wp0, wp1)
    x0 = x0p.reshape(NPAD, DH)
    x1 = x1p.reshape(NPAD, DH)

    ei3 = edge_index.reshape(2, nchunk, CH)

    ef0, ef1, dinv = _make_scatter_kernel(nchunk, 0, True)(x0, x1, ei3)

    oh0, oh1 = _make_scatter_kernel(nchunk, 1, False)(ef0, ef1, ei3)

    ob = 1000
    dinv_col = dinv.reshape(NPAD, 1)
    bias2 = bias.reshape(1, d_out)
    out = pl.pallas_call(
        _out_body,
        grid=(n // ob,),
        in_specs=[
            pl.BlockSpec((ob, DH), lambda i: (i, 0)),
            pl.BlockSpec((ob, DH), lambda i: (i, 0)),
            pl.BlockSpec((ob, 1), lambda i: (i, 0)),
            pl.BlockSpec((1, d_out), lambda i: (0, 0)),
        ],
        out_specs=pl.BlockSpec((ob, d_out), lambda i: (i, 0)),
        out_shape=jax.ShapeDtypeStruct((n, d_out), jnp.float32),
    )(oh0, oh1, dinv_col, bias2)
    return out
